# Initial kernel scaffold; baseline (speedup 1.0000x reference)
import jax
import jax.numpy as jnp
from jax.experimental import pallas as pl


def kernel(x, xyz, Wq, Wkv, Wproj, Wr1, bn_r_g, bn_r_b, Wr2, n1_g, n1_b, n2_g, n2_b, Wf1, bf1, Wf2, bf2):
    raise NotImplementedError("write your pallas kernel here")


# speedup vs baseline: 10.2793x; 10.2793x over previous
import functools

import jax
import jax.numpy as jnp
from jax import lax
from jax.experimental import pallas as pl
from jax.experimental.pallas import tpu as pltpu
from jax.experimental.pallas import tpu_sc as plsc

B, C, N, H, K = 2, 256, 2048, 8, 16
D = C // H
PW = 16
PWG = 128
BN = B * N
BNK = BN * K
SCALE = D ** -0.5
EPS = 1e-5
TN = 512
RK = 256
TM = 128
TF = 512
NW = 32
CH = 128
F32 = jnp.float32


def _dot(a, b, ca, cb, precision=None):
    return lax.dot_general(a, b, (((ca,), (cb,)), ((), ())),
                           preferred_element_type=F32, precision=precision)


def _prep_body(x_ref, xyz_ref, wq_ref, wkv_ref,
               xtab_ref, qtab_ref, ktab_ref, vtab_ref, ptab_ref, ptabw_ref):
    xT = x_ref[0].T
    xtab_ref[...] = xT
    qtab_ref[...] = _dot(xT, wq_ref[...], 1, 1)
    kv = _dot(xT, wkv_ref[...], 1, 1)
    ktab_ref[...] = kv[:, :C]
    vtab_ref[...] = kv[:, C:]
    p16 = jnp.concatenate(
        [xyz_ref[0], jnp.zeros((PW - 3, TN), F32)], axis=0)
    pT = p16.T
    ptab_ref[...] = pT
    ptabw_ref[...] = jnp.concatenate(
        [pT, jnp.zeros((TN, PWG - PW), F32)], axis=1)


def _prep(x, xyz, Wq, Wkv):
    nj = N // TN
    return pl.pallas_call(
        _prep_body,
        grid=(B, nj),
        in_specs=[
            pl.BlockSpec((1, C, TN), lambda b, j: (b, 0, j)),
            pl.BlockSpec((1, 3, TN), lambda b, j: (b, 0, j)),
            pl.BlockSpec((C, C), lambda b, j: (0, 0)),
            pl.BlockSpec((2 * C, C), lambda b, j: (0, 0)),
        ],
        out_specs=[
            pl.BlockSpec((TN, C), lambda b, j: (b * (N // TN) + j, 0)),
            pl.BlockSpec((TN, C), lambda b, j: (b * (N // TN) + j, 0)),
            pl.BlockSpec((TN, C), lambda b, j: (b * (N // TN) + j, 0)),
            pl.BlockSpec((TN, C), lambda b, j: (b * (N // TN) + j, 0)),
            pl.BlockSpec((TN, PW), lambda b, j: (b * (N // TN) + j, 0)),
            pl.BlockSpec((TN, PWG), lambda b, j: (b * (N // TN) + j, 0)),
        ],
        out_shape=[
            jax.ShapeDtypeStruct((BN, C), F32),
            jax.ShapeDtypeStruct((BN, C), F32),
            jax.ShapeDtypeStruct((BN, C), F32),
            jax.ShapeDtypeStruct((BN, C), F32),
            jax.ShapeDtypeStruct((BN, PW), F32),
            jax.ShapeDtypeStruct((BN, PWG), F32),
        ],
    )(x, xyz, Wq, Wkv)


def _knn_body(pblk_ref, pall_ref, idx_ref):
    pb = pblk_ref[...]
    pa = pall_ref[...]
    g = _dot(pb, pa, 1, 1)
    sqa = _dot(jnp.ones((1, PW), F32), pa * pa, 1, 1,
               precision=lax.Precision.HIGHEST)
    d2 = sqa - 2.0 * g
    iota = lax.broadcasted_iota(jnp.int32, (RK, N), 1)
    cols = []
    for _ in range(K):
        m = jnp.min(d2, axis=1, keepdims=True)
        cand = jnp.min(jnp.where(d2 <= m, iota, N), axis=1, keepdims=True)
        cols.append(cand)
        d2 = jnp.where(iota == cand, 3.0e38, d2)
    b = pl.program_id(0)
    idx_ref[...] = jnp.concatenate(cols, axis=1) + b * N


def _knn(ptab):
    nj = N // RK
    return pl.pallas_call(
        _knn_body,
        grid=(B, nj),
        in_specs=[
            pl.BlockSpec((RK, PW), lambda b, j: (b * (N // RK) + j, 0)),
            pl.BlockSpec((N, PW), lambda b, j: (b, 0)),
        ],
        out_specs=pl.BlockSpec((RK, K), lambda b, j: (b * (N // RK) + j, 0)),
        out_shape=jax.ShapeDtypeStruct((BN, K), jnp.int32),
    )(ptab, ptab)


def _gather_sc(idx2d, ktab, vtab, ptab):
    nch = BNK // CH // NW

    mesh = plsc.VectorSubcoreMesh(core_axis_name="c", subcore_axis_name="s")

    @functools.partial(
        pl.kernel,
        mesh=mesh,
        out_type=[
            jax.ShapeDtypeStruct((BNK, C), F32),
            jax.ShapeDtypeStruct((BNK, C), F32),
            jax.ShapeDtypeStruct((BNK, PWG), F32),
        ],
        scratch_types=[
            pltpu.VMEM((nch, CH), jnp.int32),
            pltpu.VMEM((CH, C), F32),
            pltpu.VMEM((CH, C), F32),
            pltpu.VMEM((CH, PWG), F32),
            pltpu.SemaphoreType.DMA,
        ],
    )
    def gather(idx_hbm, k_hbm, v_hbm, p_hbm, kg_hbm, vg_hbm, pg_hbm,
               idx_v, kbuf, vbuf, pbuf, sem):
        cid = lax.axis_index("c")
        sid = lax.axis_index("s")
        wid = sid * 2 + cid
        pltpu.sync_copy(idx_hbm.at[pl.ds(wid * nch, nch)], idx_v)

        def body(ci, _):
            irow = idx_v.at[ci]
            ck = pltpu.async_copy(k_hbm.at[irow], kbuf, sem)
            cv = pltpu.async_copy(v_hbm.at[irow], vbuf, sem)
            cp = pltpu.async_copy(p_hbm.at[irow], pbuf, sem)
            ck.wait()
            cv.wait()
            cp.wait()
            row0 = pl.multiple_of((wid * nch + ci) * CH, CH)
            pltpu.sync_copy(kbuf, kg_hbm.at[pl.ds(row0, CH)])
            pltpu.sync_copy(vbuf, vg_hbm.at[pl.ds(row0, CH)])
            pltpu.sync_copy(pbuf, pg_hbm.at[pl.ds(row0, CH)])
            return 0

        lax.fori_loop(0, nch, body, 0)

    return gather(idx2d, ktab, vtab, ptab)


def _logmom_body(q_ref, kg_ref, pg_ref, pb_ref, s_ref,
                 lg_ref, ms_ref, mm_ref):
    q = q_ref[...]
    kg3 = kg_ref[...].reshape(TM, K, C)
    prod = (kg3 * q[:, None, :]).reshape(TM * K, C)
    lg = _dot(prod, s_ref[...], 1, 0) * SCALE
    lg_ref[...] = lg.reshape(TM, K, H)
    rel = pg_ref[...][:, :PW] - jnp.broadcast_to(
        pb_ref[...][:, None, :], (TM, K, PW)).reshape(TM * K, PW)
    s = jnp.sum(rel, axis=0, keepdims=True)
    m = _dot(rel, rel, 0, 0)
    i = pl.program_id(0)

    @pl.when(i == 0)
    def _():
        ms_ref[...] = s
        mm_ref[...] = m

    @pl.when(i > 0)
    def _():
        ms_ref[...] += s
        mm_ref[...] += m


def _logmom(qtab, kg, pg, ptab, smask):
    nt = BN // TM
    return pl.pallas_call(
        _logmom_body,
        grid=(nt,),
        in_specs=[
            pl.BlockSpec((TM, C), lambda i: (i, 0)),
            pl.BlockSpec((TM * K, C), lambda i: (i, 0)),
            pl.BlockSpec((TM * K, PWG), lambda i: (i, 0)),
            pl.BlockSpec((TM, PW), lambda i: (i, 0)),
            pl.BlockSpec((C, H), lambda i: (0, 0)),
        ],
        out_specs=[
            pl.BlockSpec((TM, K, H), lambda i: (i, 0, 0)),
            pl.BlockSpec((1, PW), lambda i: (0, 0)),
            pl.BlockSpec((PW, PW), lambda i: (0, 0)),
        ],
        out_shape=[
            jax.ShapeDtypeStruct((BN, K, H), F32),
            jax.ShapeDtypeStruct((1, PW), F32),
            jax.ShapeDtypeStruct((PW, PW), F32),
        ],
    )(qtab, kg, pg, ptab, smask)


def _attn_body(lg_ref, vg_ref, pg_ref, pb_ref, x_ref, s_ref,
               ms_ref, mm_ref, wr1_ref, gcol_ref, bcol_ref, wr2_ref,
               wproj_ref, r1_ref, s1_ref, ss1_ref):
    cnt = float(BNK)
    mean = ms_ref[...] / cnt
    cov = mm_ref[...] / cnt - _dot(mean, mean, 0, 0)
    w1 = wr1_ref[...]
    w1p = jnp.concatenate([w1, jnp.zeros((C, PW - 3), F32)], axis=1)
    mu = _dot(w1p, mean, 1, 1)
    wc = _dot(w1p, cov, 1, 1)
    var = jnp.sum(wc * w1p, axis=1, keepdims=True)
    sA = gcol_ref[...] * lax.rsqrt(var + EPS)
    shift = bcol_ref[...] - mu * sA
    weff = jnp.concatenate(
        [w1 * sA, shift, jnp.zeros((C, PW - 4), F32)], axis=1)

    rel = pg_ref[...][:, :PW] - jnp.broadcast_to(
        pb_ref[...][:, None, :], (TM, K, PW)).reshape(TM * K, PW)
    lane = lax.broadcasted_iota(jnp.int32, (1, PW), 1)
    rel = rel + jnp.where(lane == 3, 1.0, 0.0)

    h1 = jnp.maximum(_dot(rel, weff, 1, 1), 0.0)
    bias = _dot(h1, wr2_ref[...], 1, 1)
    biask = _dot(bias, s_ref[...], 1, 0).reshape(TM, K, H)

    t = lg_ref[...] + biask
    mx = jnp.max(t, axis=1, keepdims=True)
    e = jnp.exp(t - mx)
    a = e / jnp.sum(e, axis=1, keepdims=True)
    aexp = _dot(a.reshape(TM * K, H), s_ref[...], 1, 1)
    wv = (aexp * vg_ref[...]).reshape(TM, K, C)
    attnout = jnp.sum(wv, axis=1)
    out = _dot(attnout, wproj_ref[...], 1, 1)
    r1 = x_ref[...] + out
    r1_ref[...] = r1
    s1 = jnp.sum(r1, axis=0, keepdims=True)
    ss1 = jnp.sum(r1 * r1, axis=0, keepdims=True)
    i = pl.program_id(0)

    @pl.when(i == 0)
    def _():
        s1_ref[...] = s1
        ss1_ref[...] = ss1

    @pl.when(i > 0)
    def _():
        s1_ref[...] += s1
        ss1_ref[...] += ss1


def _attn(lg, vg, pg, ptab, xtab, smask, ms, mm, Wr1, gcol, bcol, Wr2, Wproj):
    nt = BN // TM
    return pl.pallas_call(
        _attn_body,
        grid=(nt,),
        in_specs=[
            pl.BlockSpec((TM, K, H), lambda i: (i, 0, 0)),
            pl.BlockSpec((TM * K, C), lambda i: (i, 0)),
            pl.BlockSpec((TM * K, PWG), lambda i: (i, 0)),
            pl.BlockSpec((TM, PW), lambda i: (i, 0)),
            pl.BlockSpec((TM, C), lambda i: (i, 0)),
            pl.BlockSpec((C, H), lambda i: (0, 0)),
            pl.BlockSpec((1, PW), lambda i: (0, 0)),
            pl.BlockSpec((PW, PW), lambda i: (0, 0)),
            pl.BlockSpec((C, 3), lambda i: (0, 0)),
            pl.BlockSpec((C, 1), lambda i: (0, 0)),
            pl.BlockSpec((C, 1), lambda i: (0, 0)),
            pl.BlockSpec((C, C), lambda i: (0, 0)),
            pl.BlockSpec((C, C), lambda i: (0, 0)),
        ],
        out_specs=[
            pl.BlockSpec((TM, C), lambda i: (i, 0)),
            pl.BlockSpec((1, C), lambda i: (0, 0)),
            pl.BlockSpec((1, C), lambda i: (0, 0)),
        ],
        out_shape=[
            jax.ShapeDtypeStruct((BN, C), F32),
            jax.ShapeDtypeStruct((1, C), F32),
            jax.ShapeDtypeStruct((1, C), F32),
        ],
    )(lg, vg, pg, ptab, xtab, smask, ms, mm, Wr1, gcol, bcol, Wr2, Wproj)


def _ffn_body(r1_ref, s1_ref, ss1_ref, wf1_ref, bf1_ref, wf2_ref, bf2_ref,
              g1_ref, b1_ref, r2_ref, s2_ref, ss2_ref):
    cnt = float(BN)
    mean = s1_ref[...] / cnt
    var = ss1_ref[...] / cnt - mean * mean
    rstd = lax.rsqrt(var + EPS)
    x1 = (r1_ref[...] - mean) * rstd * g1_ref[...] + b1_ref[...]
    f = jnp.maximum(_dot(x1, wf1_ref[...], 1, 1) + bf1_ref[...], 0.0)
    f2 = _dot(f, wf2_ref[...], 1, 1) + bf2_ref[...]
    r2 = x1 + f2
    r2_ref[...] = r2
    s2 = jnp.sum(r2, axis=0, keepdims=True)
    ss2 = jnp.sum(r2 * r2, axis=0, keepdims=True)
    i = pl.program_id(0)

    @pl.when(i == 0)
    def _():
        s2_ref[...] = s2
        ss2_ref[...] = ss2

    @pl.when(i > 0)
    def _():
        s2_ref[...] += s2
        ss2_ref[...] += ss2


def _ffn(r1, s1, ss1, Wf1, bf1row, Wf2, bf2row, g1row, b1row):
    nt = BN // TF
    C4 = 4 * C
    return pl.pallas_call(
        _ffn_body,
        grid=(nt,),
        in_specs=[
            pl.BlockSpec((TF, C), lambda i: (i, 0)),
            pl.BlockSpec((1, C), lambda i: (0, 0)),
            pl.BlockSpec((1, C), lambda i: (0, 0)),
            pl.BlockSpec((C4, C), lambda i: (0, 0)),
            pl.BlockSpec((1, C4), lambda i: (0, 0)),
            pl.BlockSpec((C, C4), lambda i: (0, 0)),
            pl.BlockSpec((1, C), lambda i: (0, 0)),
            pl.BlockSpec((1, C), lambda i: (0, 0)),
            pl.BlockSpec((1, C), lambda i: (0, 0)),
        ],
        out_specs=[
            pl.BlockSpec((TF, C), lambda i: (i, 0)),
            pl.BlockSpec((1, C), lambda i: (0, 0)),
            pl.BlockSpec((1, C), lambda i: (0, 0)),
        ],
        out_shape=[
            jax.ShapeDtypeStruct((BN, C), F32),
            jax.ShapeDtypeStruct((1, C), F32),
            jax.ShapeDtypeStruct((1, C), F32),
        ],
    )(r1, s1, ss1, Wf1, bf1row, Wf2, bf2row, g1row, b1row)


def _out_body(r2_ref, s2_ref, ss2_ref, g2_ref, b2_ref, o_ref):
    cnt = float(BN)
    mean = s2_ref[...] / cnt
    var = ss2_ref[...] / cnt - mean * mean
    rstd = lax.rsqrt(var + EPS)
    x2 = (r2_ref[...] - mean) * rstd * g2_ref[...] + b2_ref[...]
    o_ref[...] = x2.T[None]


def _outk(r2, s2, ss2, g2row, b2row):
    nj = N // TF
    return pl.pallas_call(
        _out_body,
        grid=(B, nj),
        in_specs=[
            pl.BlockSpec((TF, C), lambda b, j: (b * (N // TF) + j, 0)),
            pl.BlockSpec((1, C), lambda b, j: (0, 0)),
            pl.BlockSpec((1, C), lambda b, j: (0, 0)),
            pl.BlockSpec((1, C), lambda b, j: (0, 0)),
            pl.BlockSpec((1, C), lambda b, j: (0, 0)),
        ],
        out_specs=pl.BlockSpec((1, C, TF), lambda b, j: (b, 0, j)),
        out_shape=jax.ShapeDtypeStruct((B, C, N), F32),
    )(r2, s2, ss2, g2row, b2row)


def kernel(x, xyz, Wq, Wkv, Wproj, Wr1, bn_r_g, bn_r_b, Wr2,
           n1_g, n1_b, n2_g, n2_b, Wf1, bf1, Wf2, bf2):
    smask = (jnp.arange(C)[:, None] // D == jnp.arange(H)[None, :]).astype(F32)

    xtab, qtab, ktab, vtab, ptab, ptabw = _prep(x, xyz, Wq, Wkv)
    idx = _knn(ptab)
    kg, vg, pg = _gather_sc(idx.reshape(BNK // CH, CH), ktab, vtab, ptabw)
    lg, ms, mm = _logmom(qtab, kg, pg, ptab, smask)
    r1, s1, ss1 = _attn(lg, vg, pg, ptab, xtab, smask, ms, mm,
                        Wr1, bn_r_g.reshape(C, 1), bn_r_b.reshape(C, 1),
                        Wr2, Wproj)
    r2, s2, ss2 = _ffn(r1, s1, ss1, Wf1, bf1.reshape(1, 4 * C),
                       Wf2, bf2.reshape(1, C),
                       n1_g.reshape(1, C), n1_b.reshape(1, C))
    return _outk(r2, s2, ss2, n2_g.reshape(1, C), n2_b.reshape(1, C))
